# baseline (speedup 1.0000x reference)
import functools
import math

import numpy as np
import jax
import jax.numpy as jnp
from jax import lax
from jax.experimental import pallas as pl
from jax.experimental.pallas import tpu as pltpu
from jax.experimental.pallas import tpu_sc as plsc

_LANES = 16
_SUB = 100


def _pos_encoding(seq_len: int, d: int) -> np.ndarray:
    pos = np.arange(seq_len, dtype=np.float32)[:, None]
    fill = pos * np.exp(
        -np.arange(0, d, 2, dtype=np.float32) * math.log(10000.0) / d)
    pe = np.zeros((seq_len, d), dtype=np.float32)
    pe[:, 0::2] = np.sin(fill)
    pe[:, 1::2] = np.cos(fill)
    return pe


def kernel(x, table):
    b, seq = x.shape
    _, d = table.shape
    info = plsc.get_sparse_core_info()
    nw = info.num_cores * info.num_subcores
    seqs_w = b // nw
    n_w = seqs_w * seq
    subs_w = n_w // _SUB
    gathers_per_seq = seq // _SUB

    x_sub = x.reshape(-1).astype(jnp.int32).reshape(b * seq // _SUB, _SUB)
    pe = jnp.asarray(_pos_encoding(seq, d))

    mesh = plsc.VectorSubcoreMesh(core_axis_name="c", subcore_axis_name="s")

    @functools.partial(
        pl.kernel,
        mesh=mesh,
        out_type=jax.ShapeDtypeStruct((b * seq, d), jnp.float32),
        compiler_params=pltpu.CompilerParams(use_tc_tiling_on_sc=False),
        scratch_types=[
            pltpu.VMEM((subs_w, _SUB), jnp.int32),
            pltpu.VMEM((seq, d), jnp.float32),
            pltpu.VMEM((seq, d), jnp.float32),
            pltpu.SemaphoreType.DMA,
        ],
    )
    def run(x_hbm, table_hbm, pe_hbm, out_hbm, idx_v, pe_v, rows_v, sem):
        wid = lax.axis_index("s") * info.num_cores + lax.axis_index("c")
        pltpu.sync_copy(x_hbm.at[pl.ds(wid * subs_w, subs_w)], idx_v)
        pltpu.sync_copy(pe_hbm, pe_v)

        def seq_body(t, carry):
            handles = []
            for g in range(gathers_per_seq):
                handles.append(pltpu.async_copy(
                    table_hbm.at[idx_v.at[t * gathers_per_seq + g]],
                    rows_v.at[pl.ds(g * _SUB, _SUB)],
                    sem,
                ))
            for h in handles:
                h.wait()

            def add_body(r, c2):
                for j in range(d // _LANES):
                    plsc.addupdate(
                        rows_v.at[r, pl.ds(j * _LANES, _LANES)],
                        pe_v[r, pl.ds(j * _LANES, _LANES)],
                    )
                return c2

            lax.fori_loop(0, seq, add_body, 0, unroll=4)
            pltpu.sync_copy(rows_v, out_hbm.at[pl.ds(wid * n_w + t * seq, seq)])
            return carry

        lax.fori_loop(0, seqs_w, seq_body, 0)

    out = run(x_sub, table, pe)
    return out.reshape(b, seq, d)


# speedup vs baseline: 1.1263x; 1.1263x over previous
import functools
import math

import numpy as np
import jax
import jax.numpy as jnp
from jax import lax
from jax.experimental import pallas as pl
from jax.experimental.pallas import tpu as pltpu
from jax.experimental.pallas import tpu_sc as plsc

_LANES = 16
_SUB = 100
_SEQ_PER_CHUNK = 2


def _pos_encoding(seq_len: int, d: int) -> np.ndarray:
    pos = np.arange(seq_len, dtype=np.float32)[:, None]
    fill = pos * np.exp(
        -np.arange(0, d, 2, dtype=np.float32) * math.log(10000.0) / d)
    pe = np.zeros((seq_len, d), dtype=np.float32)
    pe[:, 0::2] = np.sin(fill)
    pe[:, 1::2] = np.cos(fill)
    return pe


def kernel(x, table):
    b, seq = x.shape
    _, d = table.shape
    info = plsc.get_sparse_core_info()
    nw = info.num_cores * info.num_subcores
    seqs_w = b // nw
    n_w = seqs_w * seq
    subs_w = n_w // _SUB
    rows_c = _SEQ_PER_CHUNK * seq
    subs_c = rows_c // _SUB
    nchunks = seqs_w // _SEQ_PER_CHUNK

    x_sub = x.reshape(-1).astype(jnp.int32).reshape(b * seq // _SUB, _SUB)
    pe = jnp.asarray(np.tile(_pos_encoding(seq, d), (_SEQ_PER_CHUNK, 1)))

    mesh = plsc.VectorSubcoreMesh(core_axis_name="c", subcore_axis_name="s")

    @functools.partial(
        pl.kernel,
        mesh=mesh,
        out_type=jax.ShapeDtypeStruct((b * seq, d), jnp.float32),
        compiler_params=pltpu.CompilerParams(use_tc_tiling_on_sc=False),
        scratch_types=[
            pltpu.VMEM((subs_w, _SUB), jnp.int32),
            pltpu.VMEM((rows_c, d), jnp.float32),
            pltpu.VMEM((rows_c, d), jnp.float32),
            pltpu.VMEM((rows_c, d), jnp.float32),
            pltpu.SemaphoreType.DMA,
            pltpu.SemaphoreType.DMA,
            pltpu.SemaphoreType.DMA,
            pltpu.SemaphoreType.DMA,
        ],
    )
    def run(x_hbm, table_hbm, pe_hbm, out_hbm,
            idx_v, pe_v, rows0, rows1, gsem0, gsem1, ssem0, ssem1):
        wid = lax.axis_index("s") * info.num_cores + lax.axis_index("c")
        pltpu.sync_copy(x_hbm.at[pl.ds(wid * subs_w, subs_w)], idx_v)
        pltpu.sync_copy(pe_hbm, pe_v)

        bufs = (rows0, rows1)
        gsems = (gsem0, gsem1)
        ssems = (ssem0, ssem1)

        def fire_gather(c, buf, sem):
            for g in range(subs_c):
                pltpu.async_copy(
                    table_hbm.at[idx_v.at[c * subs_c + g]],
                    buf.at[pl.ds(g * _SUB, _SUB)],
                    sem,
                )

        def wait_gather(buf, sem):
            pltpu.make_async_copy(table_hbm.at[pl.ds(0, rows_c)], buf, sem).wait()

        def fire_scatter(c, buf, sem):
            pltpu.async_copy(buf, out_hbm.at[pl.ds(wid * n_w + c * rows_c, rows_c)], sem)

        def wait_scatter(buf, sem):
            pltpu.make_async_copy(
                buf, out_hbm.at[pl.ds(wid * n_w, rows_c)], sem).wait()

        fire_gather(0, rows0, gsem0)

        def chunk_pair(t, carry):
            for bslot in range(2):
                c = t + bslot
                buf, gsem, ssem = bufs[bslot], gsems[bslot], ssems[bslot]
                obuf, ogsem, ossem = bufs[1 - bslot], gsems[1 - bslot], ssems[1 - bslot]

                wait_gather(buf, gsem)

                @pl.when(c >= 1)
                def _():
                    wait_scatter(obuf, ossem)

                @pl.when(c + 1 < nchunks)
                def _():
                    fire_gather(c + 1, obuf, ogsem)

                def add_body(r, c2):
                    for j in range(d // _LANES):
                        plsc.addupdate(
                            buf.at[r, pl.ds(j * _LANES, _LANES)],
                            pe_v[r, pl.ds(j * _LANES, _LANES)],
                        )
                    return c2

                lax.fori_loop(0, rows_c, add_body, 0, unroll=8)
                fire_scatter(c, buf, ssem)
            return carry

        lax.fori_loop(0, nchunks // 2, lambda i, cr: chunk_pair(i * 2, cr), 0)
        wait_scatter(bufs[1], ssems[1])

    out = run(x_sub, table, pe)
    return out.reshape(b, seq, d)


# speedup vs baseline: 1.1265x; 1.0002x over previous
import functools
import math

import numpy as np
import jax
import jax.numpy as jnp
from jax import lax
from jax.experimental import pallas as pl
from jax.experimental.pallas import tpu as pltpu
from jax.experimental.pallas import tpu_sc as plsc

_LANES = 16
_SUB = 100
_SEQ_PER_CHUNK = 2


def _pos_encoding(seq_len: int, d: int) -> np.ndarray:
    pos = np.arange(seq_len, dtype=np.float32)[:, None]
    fill = pos * np.exp(
        -np.arange(0, d, 2, dtype=np.float32) * math.log(10000.0) / d)
    pe = np.zeros((seq_len, d), dtype=np.float32)
    pe[:, 0::2] = np.sin(fill)
    pe[:, 1::2] = np.cos(fill)
    return pe


def kernel(x, table):
    b, seq = x.shape
    _, d = table.shape
    info = plsc.get_sparse_core_info()
    nw = info.num_cores * info.num_subcores
    seqs_w = b // nw
    subs_seq = seq // _SUB
    nchunks = seqs_w // _SEQ_PER_CHUNK

    x3 = x.astype(jnp.int32).reshape(b, subs_seq, _SUB)
    pe = jnp.asarray(_pos_encoding(seq, d))

    mesh = plsc.VectorSubcoreMesh(core_axis_name="c", subcore_axis_name="s")

    @functools.partial(
        pl.kernel,
        mesh=mesh,
        out_type=jax.ShapeDtypeStruct((b, seq, d), jnp.float32),
        compiler_params=pltpu.CompilerParams(use_tc_tiling_on_sc=False),
        scratch_types=[
            pltpu.VMEM((seqs_w, subs_seq, _SUB), jnp.int32),
            pltpu.VMEM((seq, d), jnp.float32),
            pltpu.VMEM((_SEQ_PER_CHUNK, seq, d), jnp.float32),
            pltpu.VMEM((_SEQ_PER_CHUNK, seq, d), jnp.float32),
            pltpu.SemaphoreType.DMA,
            pltpu.SemaphoreType.DMA,
            pltpu.SemaphoreType.DMA,
            pltpu.SemaphoreType.DMA,
        ],
    )
    def run(x_hbm, table_hbm, pe_hbm, out_hbm,
            idx_v, pe_v, rows0, rows1, gsem0, gsem1, ssem0, ssem1):
        wid = lax.axis_index("s") * info.num_cores + lax.axis_index("c")
        row0 = wid * seqs_w
        pltpu.sync_copy(x_hbm.at[pl.ds(row0, seqs_w)], idx_v)
        pltpu.sync_copy(pe_hbm, pe_v)

        bufs = (rows0, rows1)
        gsems = (gsem0, gsem1)
        ssems = (ssem0, ssem1)

        def fire_gather(c, buf, sem):
            for j in range(_SEQ_PER_CHUNK):
                for g in range(subs_seq):
                    pltpu.async_copy(
                        table_hbm.at[idx_v.at[c * _SEQ_PER_CHUNK + j, g]],
                        buf.at[j, pl.ds(g * _SUB, _SUB)],
                        sem,
                    )

        def wait_gather(buf, sem):
            for j in range(_SEQ_PER_CHUNK):
                pltpu.make_async_copy(
                    table_hbm.at[pl.ds(0, seq)], buf.at[j], sem).wait()

        def fire_scatter(c, buf, sem):
            pltpu.async_copy(
                buf, out_hbm.at[pl.ds(row0 + c * _SEQ_PER_CHUNK,
                                      _SEQ_PER_CHUNK)], sem)

        def wait_scatter(buf, sem):
            pltpu.make_async_copy(
                buf, out_hbm.at[pl.ds(row0, _SEQ_PER_CHUNK)], sem).wait()

        fire_gather(0, rows0, gsem0)

        def chunk_pair(t, carry):
            for bslot in range(2):
                c = t + bslot
                buf, gsem, ssem = bufs[bslot], gsems[bslot], ssems[bslot]
                obuf, ogsem, ossem = (bufs[1 - bslot], gsems[1 - bslot],
                                      ssems[1 - bslot])

                wait_gather(buf, gsem)

                @pl.when(c >= 1)
                def _():
                    wait_scatter(obuf, ossem)

                @pl.when(c + 1 < nchunks)
                def _():
                    fire_gather(c + 1, obuf, ogsem)

                for j in range(_SEQ_PER_CHUNK):
                    def add_body(r, c2, j=j):
                        for k in range(d // _LANES):
                            plsc.addupdate(
                                buf.at[j, r, pl.ds(k * _LANES, _LANES)],
                                pe_v[r, pl.ds(k * _LANES, _LANES)],
                            )
                        return c2

                    lax.fori_loop(0, seq, add_body, 0, unroll=8)
                fire_scatter(c, buf, ssem)
            return carry

        lax.fori_loop(0, nchunks // 2, lambda i, cr: chunk_pair(i * 2, cr), 0)
        wait_scatter(bufs[1], ssems[1])

    return run(x3, table, pe)
